# baseline (speedup 1.0000x reference)
import functools

import jax
import jax.numpy as jnp
from jax import lax
from jax.experimental import pallas as pl
from jax.experimental.pallas import tpu as pltpu
from jax.experimental.pallas import tpu_sc as plsc

B, C, H, W = 4, 3136, 56, 56
HB = 8
NH = H // HB
K = 64
NPOS = B * H * W
NWORK = 32
RPW = NPOS // NWORK
NCH = RPW // 8


def _reduce_body(x_ref, ma_ref, sel_ref, vacc):
    i = pl.program_id(1)
    x = x_ref[0]
    mx = jnp.max(x, axis=2)
    sm = jnp.sum(x, axis=2) * (1.0 / C)
    ma_ref[0] = jnp.stack([mx, sm], axis=-1)
    pv = jnp.sum(x, axis=(0, 1))[None, :]

    @pl.when(i == 0)
    def _():
        vacc[...] = pv

    @pl.when(i > 0)
    def _():
        vacc[...] = vacc[...] + pv

    @pl.when(i == NH - 1)
    def _():
        lanes = lax.broadcasted_iota(jnp.int32, (1, C), 1)
        lanes_k = lax.broadcasted_iota(jnp.int32, (1, K), 1)

        def body(j, carry):
            v, selv = carry
            m = jnp.max(v)
            idx = jnp.min(jnp.where(v >= m, lanes, C))
            selv = jnp.where(lanes_k == j, idx, selv)
            v = jnp.where(lanes == idx, -jnp.inf, v)
            return v, selv

        _, selv = lax.fori_loop(0, K, body,
                                (vacc[...], jnp.zeros((1, K), jnp.int32)))
        sel_ref[0] = selv


def _sc_gather_body(corr_hbm, ma_hbm, sel_hbm, out_hbm, sel_v, xa_v, xb_v,
                    ma_v, o_v, sa, sb):
    w = lax.axis_index("s") * 2 + lax.axis_index("c")
    b = w // 8
    base = w * RPW
    pltpu.sync_copy(sel_hbm, sel_v)
    iota = lax.broadcasted_iota(jnp.int32, (16,), 0)
    cidx = [sel_v[pl.ds(b * K + k * 16, 16)] for k in range(K // 16)]

    for g in range(RPW // 56):
        pltpu.sync_copy(ma_hbm.at[pl.ds(base + g * 56, 56)], ma_v)
        for q in range(7):
            mvals = plsc.load_gather(ma_v, [q * 8 + (iota >> 1), iota & 1])
            plsc.store_scatter(
                o_v, [g * 56 + q * 8 + (iota >> 1), iota & 1], mvals)

    def start(m, buf, sem):
        pltpu.make_async_copy(
            corr_hbm.at[pl.ds(base + m * 8, 8)], buf, sem).start()

    def finish(m, buf, sem):
        pltpu.make_async_copy(
            corr_hbm.at[pl.ds(base + m * 8, 8)], buf, sem).wait()
        for k in range(K // 16):
            oidx = 2 + k * 16 + iota
            for s in range(8):
                vals = plsc.load_gather(buf, [jnp.full((16,), s, jnp.int32),
                                              cidx[k]])
                plsc.store_scatter(
                    o_v, [jnp.full((16,), s, jnp.int32) + m * 8, oidx], vals)

    def step(m, carry):
        @pl.when(m < NCH - 1)
        def _():
            @pl.when(m % 2 == 0)
            def _():
                start(m + 1, xb_v, sb)

            @pl.when(m % 2 == 1)
            def _():
                start(m + 1, xa_v, sa)

        @pl.when(m % 2 == 0)
        def _():
            finish(m, xa_v, sa)

        @pl.when(m % 2 == 1)
        def _():
            finish(m, xb_v, sb)

        return carry

    start(0, xa_v, sa)
    lax.fori_loop(0, NCH, step, 0)
    pltpu.sync_copy(o_v, out_hbm.at[pl.ds(base, RPW)])


@functools.cache
def _sc_gather():
    return pl.kernel(
        _sc_gather_body,
        out_type=jax.ShapeDtypeStruct((NPOS, 2 + K), jnp.float32),
        mesh=plsc.VectorSubcoreMesh(core_axis_name="c", subcore_axis_name="s"),
        scratch_types=[
            pltpu.VMEM((B * K,), jnp.int32),
            pltpu.VMEM((8, C), jnp.float32),
            pltpu.VMEM((8, C), jnp.float32),
            pltpu.VMEM((56, 2), jnp.float32),
            pltpu.VMEM((RPW, 2 + K), jnp.float32),
            pltpu.SemaphoreType.DMA,
            pltpu.SemaphoreType.DMA,
        ],
        compiler_params=pltpu.CompilerParams(needs_layout_passes=False),
    )


@jax.jit
def kernel(corr, select_indices):
    corr_t = jnp.transpose(corr, (0, 2, 3, 1))
    ma, sel = pl.pallas_call(
        _reduce_body,
        grid=(B, NH),
        in_specs=[pl.BlockSpec((1, HB, W, C), lambda b, i: (b, i, 0, 0))],
        out_specs=[
            pl.BlockSpec((1, HB, W, 2), lambda b, i: (b, i, 0, 0)),
            pl.BlockSpec((1, 1, K), lambda b, i: (b, 0, 0)),
        ],
        out_shape=[
            jax.ShapeDtypeStruct((B, H, W, 2), jnp.float32),
            jax.ShapeDtypeStruct((B, 1, K), jnp.int32),
        ],
        scratch_shapes=[pltpu.VMEM((1, C), jnp.float32)],
    )(corr_t)

    sel = jnp.take(sel.reshape(B, K), select_indices, axis=1)

    out2 = _sc_gather()(corr_t.reshape(NPOS, C), ma.reshape(NPOS, 2),
                        sel.reshape(B * K).astype(jnp.int32))
    out_t = out2.reshape(B, H, W, 2 + K)
    return jnp.transpose(out_t, (0, 3, 1, 2))


# speedup vs baseline: 1.4732x; 1.4732x over previous
import functools

import jax
import jax.numpy as jnp
from jax import lax
from jax.experimental import pallas as pl
from jax.experimental.pallas import tpu as pltpu
from jax.experimental.pallas import tpu_sc as plsc

B, C, H, W = 4, 3136, 56, 56
HB = 14
NH = H // HB
K = 64
NPOS = B * H * W
NWORK = 32
RPW = NPOS // NWORK
NCH = RPW // 8


def _reduce_body(x_ref, ma_ref, val_ref):
    i = pl.program_id(1)
    x = x_ref[0]
    mx = jnp.max(x, axis=2)
    sm = jnp.sum(x, axis=2) * (1.0 / C)
    ma_ref[0] = jnp.stack([mx, sm], axis=-1)
    pv = jnp.sum(x, axis=(0, 1))[None, :]

    @pl.when(i == 0)
    def _():
        val_ref[0] = pv

    @pl.when(i > 0)
    def _():
        val_ref[0] = val_ref[0] + pv


def _topk_body(val_ref, sel_ref):
    v = val_ref[:, 0, :]
    lanes = lax.broadcasted_iota(jnp.int32, (B, C), 1)
    lanes_k = lax.broadcasted_iota(jnp.int32, (B, K), 1)

    def body(j, carry):
        v, selv = carry
        m = jnp.max(v, axis=1, keepdims=True)
        idx = jnp.min(jnp.where(v >= m, lanes, C), axis=1,
                      keepdims=True)
        selv = jnp.where(lanes_k == j, idx, selv)
        v = jnp.where(lanes == idx, -jnp.inf, v)
        return v, selv

    _, selv = lax.fori_loop(0, K, body,
                            (v, jnp.zeros((B, K), jnp.int32)))
    sel_ref[...] = selv


def _sc_gather_body(corr_hbm, ma_hbm, sel_hbm, out_hbm, sel_v, xa_v, xb_v,
                    ma_v, o_v, sa, sb):
    w = lax.axis_index("s") * 2 + lax.axis_index("c")
    b = w // 8
    base = w * RPW
    pltpu.sync_copy(sel_hbm, sel_v)
    iota = lax.broadcasted_iota(jnp.int32, (16,), 0)
    ma_s = iota >> 1
    ma_c = iota & 1
    cidx = [sel_v[pl.ds(b * K + k * 16, 16)] for k in range(K // 16)]

    def start(m, buf, sem):
        pltpu.make_async_copy(
            corr_hbm.at[pl.ds(base + m * 8, 8)], buf, sem).start()

    def finish(m, buf, sem):
        pltpu.make_async_copy(
            corr_hbm.at[pl.ds(base + m * 8, 8)], buf, sem).wait()
        g = m % 7
        for k in range(K // 16):
            oidx = 2 + k * 16 + iota
            for s in range(8):
                sidx = jnp.full((16,), g * 8 + s, jnp.int32)
                vals = plsc.load_gather(buf, [jnp.full((16,), s, jnp.int32),
                                              cidx[k]])
                plsc.store_scatter(o_v, [sidx, oidx], vals)

    def step(m, carry):
        @pl.when(m % 7 == 0)
        def _():
            pltpu.sync_copy(ma_hbm.at[pl.ds(base + (m // 7) * 56, 56)], ma_v)
            for q in range(7):
                mvals = plsc.load_gather(ma_v, [q * 8 + ma_s, ma_c])
                plsc.store_scatter(o_v, [q * 8 + ma_s, ma_c], mvals)

        @pl.when(m < NCH - 1)
        def _():
            @pl.when(m % 2 == 0)
            def _():
                start(m + 1, xb_v, sb)

            @pl.when(m % 2 == 1)
            def _():
                start(m + 1, xa_v, sa)

        @pl.when(m % 2 == 0)
        def _():
            finish(m, xa_v, sa)

        @pl.when(m % 2 == 1)
        def _():
            finish(m, xb_v, sb)

        @pl.when(m % 7 == 6)
        def _():
            pltpu.sync_copy(o_v, out_hbm.at[pl.ds(base + (m // 7) * 56, 56)])
        return carry

    start(0, xa_v, sa)
    lax.fori_loop(0, NCH, step, 0)


@functools.cache
def _sc_gather():
    return pl.kernel(
        _sc_gather_body,
        out_type=jax.ShapeDtypeStruct((NPOS, 2 + K), jnp.float32),
        mesh=plsc.VectorSubcoreMesh(core_axis_name="c", subcore_axis_name="s"),
        scratch_types=[
            pltpu.VMEM((B * K,), jnp.int32),
            pltpu.VMEM((8, C), jnp.float32),
            pltpu.VMEM((8, C), jnp.float32),
            pltpu.VMEM((56, 2), jnp.float32),
            pltpu.VMEM((56, 2 + K), jnp.float32),
            pltpu.SemaphoreType.DMA,
            pltpu.SemaphoreType.DMA,
        ],
        compiler_params=pltpu.CompilerParams(needs_layout_passes=False),
    )


@jax.jit
def kernel(corr, select_indices):
    corr_t = jnp.transpose(corr, (0, 2, 3, 1))
    ma, val = pl.pallas_call(
        _reduce_body,
        grid=(B, NH),
        in_specs=[pl.BlockSpec((1, HB, W, C), lambda b, i: (b, i, 0, 0))],
        out_specs=[
            pl.BlockSpec((1, HB, W, 2), lambda b, i: (b, i, 0, 0)),
            pl.BlockSpec((1, 1, C), lambda b, i: (b, 0, 0)),
        ],
        out_shape=[
            jax.ShapeDtypeStruct((B, H, W, 2), jnp.float32),
            jax.ShapeDtypeStruct((B, 1, C), jnp.float32),
        ],
    )(corr_t)

    sel = pl.pallas_call(
        _topk_body,
        grid=(1,),
        in_specs=[pl.BlockSpec((B, 1, C), lambda i: (0, 0, 0))],
        out_specs=pl.BlockSpec((B, K), lambda i: (0, 0)),
        out_shape=jax.ShapeDtypeStruct((B, K), jnp.int32),
    )(val)

    sel = jnp.take(sel, select_indices, axis=1)

    out2 = _sc_gather()(corr_t.reshape(NPOS, C), ma.reshape(NPOS, 2),
                        sel.reshape(B * K).astype(jnp.int32))
    out_t = out2.reshape(B, H, W, 2 + K)
    return jnp.transpose(out_t, (0, 3, 1, 2))


# speedup vs baseline: 1.5783x; 1.0713x over previous
import functools

import jax
import jax.numpy as jnp
from jax import lax
from jax.experimental import pallas as pl
from jax.experimental.pallas import tpu as pltpu
from jax.experimental.pallas import tpu_sc as plsc

B, C, H, W = 4, 3136, 56, 56
HB = 14
NH = H // HB
K = 64
HT = 24
HS = H - HT
NPOS = B * H * W
NWORK = 32
RPW = HS * W // 8
NCH = RPW // 8
NGR = RPW // 56


def _reduce_body(x_ref, ma_ref, val_ref):
    i = pl.program_id(1)
    x = x_ref[0]
    mx = jnp.max(x, axis=2)
    sm = jnp.sum(x, axis=2) * (1.0 / C)
    ma_ref[0] = jnp.stack([mx, sm], axis=-1)
    pv = jnp.sum(x, axis=(0, 1))[None, :]

    @pl.when(i == 0)
    def _():
        val_ref[0] = pv

    @pl.when(i > 0)
    def _():
        val_ref[0] = val_ref[0] + pv


def _topk_body(val_ref, sel_ref):
    v = val_ref[:, 0, :]
    lanes = lax.broadcasted_iota(jnp.int32, (B, C), 1)
    lanes_k = lax.broadcasted_iota(jnp.int32, (B, K), 1)

    def body(j, carry):
        v, selv = carry
        m = jnp.max(v, axis=1, keepdims=True)
        idx = jnp.min(jnp.where(v >= m, lanes, C), axis=1,
                      keepdims=True)
        selv = jnp.where(lanes_k == j, idx, selv)
        v = jnp.where(lanes == idx, -jnp.inf, v)
        return v, selv

    _, selv = lax.fori_loop(0, K, body,
                            (v, jnp.zeros((B, K), jnp.int32)))
    sel_ref[...] = selv


def _tc_gather_body(x_ref, sel_ref, out_ref, oh):
    i = pl.program_id(1)

    @pl.when(i == 0)
    def _():
        crow = lax.broadcasted_iota(jnp.int32, (C, K), 0)
        oh[...] = jnp.where(crow == jnp.broadcast_to(sel_ref[0], (C, K)),
                            1.0, 0.0)

    x = x_ref[0]
    mx = jnp.max(x, axis=2)
    sm = jnp.sum(x, axis=2) * (1.0 / C)
    g = jax.lax.dot_general(x.reshape(8 * W, C), oh[...],
                            (((1,), (0,)), ((), ())),
                            preferred_element_type=jnp.float32)
    out_ref[0] = jnp.concatenate(
        [jnp.stack([mx, sm], axis=-1), g.reshape(8, W, K)], axis=-1)


def _sc_gather_body(corr_hbm, ma_hbm, sel_hbm, out_hbm, sel_v, xa_v, xb_v,
                    ma_v, o_v, sa, sb):
    w = lax.axis_index("s") * 2 + lax.axis_index("c")
    b = w // 8
    base = b * (H * W) + HT * W + (w % 8) * RPW
    obase = b * (HS * W) + (w % 8) * RPW
    pltpu.sync_copy(sel_hbm, sel_v)
    iota = lax.broadcasted_iota(jnp.int32, (16,), 0)
    ma_s = iota >> 1
    ma_c = iota & 1
    cidx = [sel_v[pl.ds(b * K + k * 16, 16)] for k in range(K // 16)]

    def start(m, buf, sem):
        pltpu.make_async_copy(
            corr_hbm.at[pl.ds(base + m * 8, 8)], buf, sem).start()

    def finish(m, buf, sem):
        pltpu.make_async_copy(
            corr_hbm.at[pl.ds(base + m * 8, 8)], buf, sem).wait()
        g = m % 7
        for k in range(K // 16):
            oidx = 2 + k * 16 + iota
            for s in range(8):
                sidx = jnp.full((16,), g * 8 + s, jnp.int32)
                vals = plsc.load_gather(buf, [jnp.full((16,), s, jnp.int32),
                                              cidx[k]])
                plsc.store_scatter(o_v, [sidx, oidx], vals)

    def step(m, carry):
        @pl.when(m % 7 == 0)
        def _():
            pltpu.sync_copy(ma_hbm.at[pl.ds(base + (m // 7) * 56, 56)], ma_v)
            for q in range(7):
                mvals = plsc.load_gather(ma_v, [q * 8 + ma_s, ma_c])
                plsc.store_scatter(o_v, [q * 8 + ma_s, ma_c], mvals)

        @pl.when(m < NCH - 1)
        def _():
            @pl.when(m % 2 == 0)
            def _():
                start(m + 1, xb_v, sb)

            @pl.when(m % 2 == 1)
            def _():
                start(m + 1, xa_v, sa)

        @pl.when(m % 2 == 0)
        def _():
            finish(m, xa_v, sa)

        @pl.when(m % 2 == 1)
        def _():
            finish(m, xb_v, sb)

        @pl.when(m % 7 == 6)
        def _():
            pltpu.sync_copy(o_v, out_hbm.at[pl.ds(obase + (m // 7) * 56, 56)])
        return carry

    start(0, xa_v, sa)
    lax.fori_loop(0, NCH, step, 0)


@functools.cache
def _sc_gather():
    return pl.kernel(
        _sc_gather_body,
        out_type=jax.ShapeDtypeStruct((B * HS * W, 2 + K), jnp.float32),
        mesh=plsc.VectorSubcoreMesh(core_axis_name="c", subcore_axis_name="s"),
        scratch_types=[
            pltpu.VMEM((B * K,), jnp.int32),
            pltpu.VMEM((8, C), jnp.float32),
            pltpu.VMEM((8, C), jnp.float32),
            pltpu.VMEM((56, 2), jnp.float32),
            pltpu.VMEM((56, 2 + K), jnp.float32),
            pltpu.SemaphoreType.DMA,
            pltpu.SemaphoreType.DMA,
        ],
        compiler_params=pltpu.CompilerParams(needs_layout_passes=False),
    )


@jax.jit
def kernel(corr, select_indices):
    corr_t = jnp.transpose(corr, (0, 2, 3, 1))
    ma, val = pl.pallas_call(
        _reduce_body,
        grid=(B, NH),
        in_specs=[pl.BlockSpec((1, HB, W, C), lambda b, i: (b, i, 0, 0))],
        out_specs=[
            pl.BlockSpec((1, HB, W, 2), lambda b, i: (b, i, 0, 0)),
            pl.BlockSpec((1, 1, C), lambda b, i: (b, 0, 0)),
        ],
        out_shape=[
            jax.ShapeDtypeStruct((B, H, W, 2), jnp.float32),
            jax.ShapeDtypeStruct((B, 1, C), jnp.float32),
        ],
    )(corr_t)

    sel = pl.pallas_call(
        _topk_body,
        grid=(1,),
        in_specs=[pl.BlockSpec((B, 1, C), lambda i: (0, 0, 0))],
        out_specs=pl.BlockSpec((B, K), lambda i: (0, 0)),
        out_shape=jax.ShapeDtypeStruct((B, K), jnp.int32),
    )(val)

    sel = jnp.take(sel, select_indices, axis=1)
    sel = sel.astype(jnp.int32)

    out_s = _sc_gather()(corr_t.reshape(NPOS, C), ma.reshape(NPOS, 2),
                         sel.reshape(B * K))

    out_tc = pl.pallas_call(
        _tc_gather_body,
        grid=(B, HT // 8),
        in_specs=[
            pl.BlockSpec((1, 8, W, C), lambda b, i: (b, i, 0, 0)),
            pl.BlockSpec((1, 1, K), lambda b, i: (b, 0, 0)),
        ],
        out_specs=pl.BlockSpec((1, 8, W, 2 + K), lambda b, i: (b, i, 0, 0)),
        out_shape=jax.ShapeDtypeStruct((B, HT, W, 2 + K), jnp.float32),
        scratch_shapes=[pltpu.VMEM((C, K), jnp.float32)],
    )(corr_t, sel.reshape(B, 1, K))

    out_t = jnp.concatenate(
        [out_tc, out_s.reshape(B, HS, W, 2 + K)], axis=1)
    return jnp.transpose(out_t, (0, 3, 1, 2))


# speedup vs baseline: 1.6183x; 1.0253x over previous
import functools

import jax
import jax.numpy as jnp
from jax import lax
from jax.experimental import pallas as pl
from jax.experimental.pallas import tpu as pltpu
from jax.experimental.pallas import tpu_sc as plsc

B, C, H, W = 4, 3136, 56, 56
HB = 14
NH = H // HB
K = 64
HT = 32
HS = H - HT
NPOS = B * H * W
NWORK = 32
RPW = HS * W // 8
NCH = RPW // 8
NGR = RPW // 56


def _reduce_body(x_ref, ma_ref, val_ref):
    i = pl.program_id(1)
    x = x_ref[0]
    mx = jnp.max(x, axis=2)
    sm = jnp.sum(x, axis=2) * (1.0 / C)
    ma_ref[0] = jnp.stack([mx, sm], axis=-1)
    pv = jnp.sum(x, axis=(0, 1))[None, :]

    @pl.when(i == 0)
    def _():
        val_ref[0] = pv

    @pl.when(i > 0)
    def _():
        val_ref[0] = val_ref[0] + pv


def _topk_body(val_ref, sel_ref):
    v = val_ref[:, 0, :]
    lanes = lax.broadcasted_iota(jnp.int32, (B, C), 1)
    lanes_k = lax.broadcasted_iota(jnp.int32, (B, K), 1)

    def body(j, carry):
        v, selv = carry
        m = jnp.max(v, axis=1, keepdims=True)
        idx = jnp.min(jnp.where(v >= m, lanes, C), axis=1,
                      keepdims=True)
        selv = jnp.where(lanes_k == j, idx, selv)
        v = jnp.where(lanes == idx, -jnp.inf, v)
        return v, selv

    _, selv = lax.fori_loop(0, K, body,
                            (v, jnp.zeros((B, K), jnp.int32)))
    sel_ref[...] = selv


def _tc_gather_body(x_ref, sel_ref, out_ref, oh):
    i = pl.program_id(1)

    @pl.when(i == 0)
    def _():
        crow = lax.broadcasted_iota(jnp.int32, (C, K), 0)
        oh[...] = jnp.where(crow == jnp.broadcast_to(sel_ref[0], (C, K)),
                            1.0, 0.0)

    x = x_ref[0]
    mx = jnp.max(x, axis=2)
    sm = jnp.sum(x, axis=2) * (1.0 / C)
    g = jax.lax.dot_general(x.reshape(8 * W, C), oh[...],
                            (((1,), (0,)), ((), ())),
                            preferred_element_type=jnp.float32)
    out_ref[0] = jnp.concatenate(
        [jnp.stack([mx, sm], axis=-1), g.reshape(8, W, K)], axis=-1)


def _sc_gather_body(corr_hbm, ma_hbm, sel_hbm, out_hbm, sel_v, xa_v, xb_v,
                    ma_v, o_v, sa, sb):
    w = lax.axis_index("s") * 2 + lax.axis_index("c")
    b = w // 8
    base = b * (H * W) + HT * W + (w % 8) * RPW
    obase = b * (HS * W) + (w % 8) * RPW
    pltpu.sync_copy(sel_hbm, sel_v)
    iota = lax.broadcasted_iota(jnp.int32, (16,), 0)
    ma_s = iota >> 1
    ma_c = iota & 1
    cidx = [sel_v[pl.ds(b * K + k * 16, 16)] for k in range(K // 16)]

    def start(m, buf, sem):
        pltpu.make_async_copy(
            corr_hbm.at[pl.ds(base + m * 8, 8)], buf, sem).start()

    def finish(m, buf, sem):
        pltpu.make_async_copy(
            corr_hbm.at[pl.ds(base + m * 8, 8)], buf, sem).wait()
        g = m % 7
        for k in range(K // 16):
            oidx = 2 + k * 16 + iota
            for s in range(8):
                sidx = jnp.full((16,), g * 8 + s, jnp.int32)
                vals = plsc.load_gather(buf, [jnp.full((16,), s, jnp.int32),
                                              cidx[k]])
                plsc.store_scatter(o_v, [sidx, oidx], vals)

    def step(m, carry):
        @pl.when(m % 7 == 0)
        def _():
            pltpu.sync_copy(ma_hbm.at[pl.ds(base + (m // 7) * 56, 56)], ma_v)
            for q in range(7):
                mvals = plsc.load_gather(ma_v, [q * 8 + ma_s, ma_c])
                plsc.store_scatter(o_v, [q * 8 + ma_s, ma_c], mvals)

        @pl.when(m < NCH - 1)
        def _():
            @pl.when(m % 2 == 0)
            def _():
                start(m + 1, xb_v, sb)

            @pl.when(m % 2 == 1)
            def _():
                start(m + 1, xa_v, sa)

        @pl.when(m % 2 == 0)
        def _():
            finish(m, xa_v, sa)

        @pl.when(m % 2 == 1)
        def _():
            finish(m, xb_v, sb)

        @pl.when(m % 7 == 6)
        def _():
            pltpu.sync_copy(o_v, out_hbm.at[pl.ds(obase + (m // 7) * 56, 56)])
        return carry

    start(0, xa_v, sa)
    lax.fori_loop(0, NCH, step, 0)


@functools.cache
def _sc_gather():
    return pl.kernel(
        _sc_gather_body,
        out_type=jax.ShapeDtypeStruct((B * HS * W, 2 + K), jnp.float32),
        mesh=plsc.VectorSubcoreMesh(core_axis_name="c", subcore_axis_name="s"),
        scratch_types=[
            pltpu.VMEM((B * K,), jnp.int32),
            pltpu.VMEM((8, C), jnp.float32),
            pltpu.VMEM((8, C), jnp.float32),
            pltpu.VMEM((56, 2), jnp.float32),
            pltpu.VMEM((56, 2 + K), jnp.float32),
            pltpu.SemaphoreType.DMA,
            pltpu.SemaphoreType.DMA,
        ],
        compiler_params=pltpu.CompilerParams(needs_layout_passes=False),
    )


@jax.jit
def kernel(corr, select_indices):
    corr_t = jnp.transpose(corr, (0, 2, 3, 1))
    ma, val = pl.pallas_call(
        _reduce_body,
        grid=(B, NH),
        in_specs=[pl.BlockSpec((1, HB, W, C), lambda b, i: (b, i, 0, 0))],
        out_specs=[
            pl.BlockSpec((1, HB, W, 2), lambda b, i: (b, i, 0, 0)),
            pl.BlockSpec((1, 1, C), lambda b, i: (b, 0, 0)),
        ],
        out_shape=[
            jax.ShapeDtypeStruct((B, H, W, 2), jnp.float32),
            jax.ShapeDtypeStruct((B, 1, C), jnp.float32),
        ],
    )(corr_t)

    sel = pl.pallas_call(
        _topk_body,
        grid=(1,),
        in_specs=[pl.BlockSpec((B, 1, C), lambda i: (0, 0, 0))],
        out_specs=pl.BlockSpec((B, K), lambda i: (0, 0)),
        out_shape=jax.ShapeDtypeStruct((B, K), jnp.int32),
    )(val)

    sel = jnp.take(sel, select_indices, axis=1)
    sel = sel.astype(jnp.int32)

    out_s = _sc_gather()(corr_t.reshape(NPOS, C), ma.reshape(NPOS, 2),
                         sel.reshape(B * K))

    out_tc = pl.pallas_call(
        _tc_gather_body,
        grid=(B, HT // 8),
        in_specs=[
            pl.BlockSpec((1, 8, W, C), lambda b, i: (b, i, 0, 0)),
            pl.BlockSpec((1, 1, K), lambda b, i: (b, 0, 0)),
        ],
        out_specs=pl.BlockSpec((1, 8, W, 2 + K), lambda b, i: (b, i, 0, 0)),
        out_shape=jax.ShapeDtypeStruct((B, HT, W, 2 + K), jnp.float32),
        scratch_shapes=[pltpu.VMEM((C, K), jnp.float32)],
    )(corr_t, sel.reshape(B, 1, K))

    out_t = jnp.concatenate(
        [out_tc, out_s.reshape(B, HS, W, 2 + K)], axis=1)
    return jnp.transpose(out_t, (0, 3, 1, 2))
